# baseline (speedup 1.0000x reference)
import jax
import jax.numpy as jnp
from jax.experimental import pallas as pl

_E = 64
_TOPK = 8
_BLK = 512


def _router_block(x_ref, w_ref, scores_ref, wts_ref, idx_ref):
    logits = jax.lax.dot_general(
        x_ref[...], w_ref[...], (((1,), (1,)), ((), ())),
        preferred_element_type=jnp.float32)
    m = jnp.max(logits, axis=-1, keepdims=True)
    e = jnp.exp(logits - m)
    s = e / jnp.sum(e, axis=-1, keepdims=True)
    scores_ref[...] = s

    iota = jax.lax.broadcasted_iota(jnp.int32, s.shape, 1)
    work = s
    wcols, icols = [], []
    for _ in range(_TOPK):
        mk = jnp.max(work, axis=-1, keepdims=True)
        hit = work == mk
        ik = jnp.min(jnp.where(hit, iota, _E), axis=-1, keepdims=True)
        wcols.append(mk)
        icols.append(ik)
        work = jnp.where(iota == ik, -jnp.inf, work)
    wts_ref[...] = jnp.concatenate(wcols, axis=1)
    idx_ref[...] = jnp.concatenate(icols, axis=1)


def kernel(x, W):
    sl, bs, hs = x.shape
    t = sl * bs
    xf = x.reshape(t, hs)
    grid = (t // _BLK,)
    scores, wts, idx = pl.pallas_call(
        _router_block,
        grid=grid,
        in_specs=[
            pl.BlockSpec((_BLK, hs), lambda i: (i, 0)),
            pl.BlockSpec((_E, hs), lambda i: (0, 0)),
        ],
        out_specs=[
            pl.BlockSpec((_BLK, _E), lambda i: (i, 0)),
            pl.BlockSpec((_BLK, _TOPK), lambda i: (i, 0)),
            pl.BlockSpec((_BLK, _TOPK), lambda i: (i, 0)),
        ],
        out_shape=[
            jax.ShapeDtypeStruct((t, _E), jnp.float32),
            jax.ShapeDtypeStruct((t, _TOPK), jnp.float32),
            jax.ShapeDtypeStruct((t, _TOPK), jnp.int32),
        ],
    )(xf, W)
    return scores, wts, idx


# speedup vs baseline: 1.0420x; 1.0420x over previous
import jax
import jax.numpy as jnp
from jax.experimental import pallas as pl

_E = 64
_TOPK = 8
_BLK = 1024


def _router_block(x_ref, w_ref, scores_ref, wts_ref, idx_ref):
    logits = jax.lax.dot_general(
        x_ref[...], w_ref[...], (((1,), (1,)), ((), ())),
        preferred_element_type=jnp.float32)
    m = jnp.max(logits, axis=-1, keepdims=True)
    e = jnp.exp(logits - m)
    s = e / jnp.sum(e, axis=-1, keepdims=True)
    scores_ref[...] = s

    iota = jax.lax.broadcasted_iota(jnp.int32, s.shape, 1)
    work = s
    wcols, icols = [], []
    for _ in range(_TOPK):
        mk = jnp.max(work, axis=-1, keepdims=True)
        hit = work == mk
        ik = jnp.min(jnp.where(hit, iota, _E), axis=-1, keepdims=True)
        wcols.append(mk)
        icols.append(ik)
        work = jnp.where(iota == ik, -jnp.inf, work)
    wts_ref[...] = jnp.concatenate(wcols, axis=1)
    idx_ref[...] = jnp.concatenate(icols, axis=1)


def kernel(x, W):
    sl, bs, hs = x.shape
    t = sl * bs
    xf = x.reshape(t, hs)
    grid = (t // _BLK,)
    scores, wts, idx = pl.pallas_call(
        _router_block,
        grid=grid,
        in_specs=[
            pl.BlockSpec((_BLK, hs), lambda i: (i, 0)),
            pl.BlockSpec((_E, hs), lambda i: (0, 0)),
        ],
        out_specs=[
            pl.BlockSpec((_BLK, _E), lambda i: (i, 0)),
            pl.BlockSpec((_BLK, _TOPK), lambda i: (i, 0)),
            pl.BlockSpec((_BLK, _TOPK), lambda i: (i, 0)),
        ],
        out_shape=[
            jax.ShapeDtypeStruct((t, _E), jnp.float32),
            jax.ShapeDtypeStruct((t, _TOPK), jnp.float32),
            jax.ShapeDtypeStruct((t, _TOPK), jnp.int32),
        ],
    )(xf, W)
    return scores, wts, idx
